# baseline (speedup 1.0000x reference)
import jax
import jax.numpy as jnp
from jax.experimental import pallas as pl

NUM_ANCHORS = 4
NUM_LABELS = 5
BLK = 1024


def _fused_heads_kernel(x_ref, w_ref, b_ref, out_ref):
    out_ref[...] = (
        jnp.dot(x_ref[...], w_ref[...], preferred_element_type=jnp.float32)
        + b_ref[...]
    )


def kernel(hidden_states, W_conf, b_conf, W_cls, b_cls, W_reg, b_reg):
    bsz, seq_len, hidden = hidden_states.shape
    x = hidden_states.reshape(bsz * seq_len, hidden)
    n = bsz * seq_len

    w = jnp.concatenate([W_conf, W_cls, W_reg], axis=1)
    b = jnp.concatenate([b_conf, b_cls, b_reg], axis=0).reshape(1, -1)
    c = w.shape[1]

    out = pl.pallas_call(
        _fused_heads_kernel,
        grid=(n // BLK,),
        in_specs=[
            pl.BlockSpec((BLK, hidden), lambda i: (i, 0)),
            pl.BlockSpec((hidden, c), lambda i: (0, 0)),
            pl.BlockSpec((1, c), lambda i: (0, 0)),
        ],
        out_specs=pl.BlockSpec((BLK, c), lambda i: (i, 0)),
        out_shape=jax.ShapeDtypeStruct((n, c), jnp.float32),
    )(x, w, b)

    conf = out[:, :NUM_ANCHORS].reshape(bsz, seq_len, NUM_ANCHORS)
    cls_ = out[:, NUM_ANCHORS:NUM_ANCHORS + NUM_ANCHORS * NUM_LABELS].reshape(
        bsz, seq_len, NUM_ANCHORS, NUM_LABELS
    )
    reg = out[:, NUM_ANCHORS + NUM_ANCHORS * NUM_LABELS:].reshape(
        bsz, seq_len, NUM_ANCHORS, 2
    )
    return (conf, cls_, reg)


# speedup vs baseline: 1.1312x; 1.1312x over previous
import jax
import jax.numpy as jnp
from jax.experimental import pallas as pl

NUM_ANCHORS = 4
NUM_LABELS = 5
BLK = 4096


def _fused_heads_kernel(x_ref, w_ref, b_ref, out_ref):
    out_ref[...] = (
        jnp.dot(x_ref[...], w_ref[...], preferred_element_type=jnp.float32)
        + b_ref[...]
    )


def kernel(hidden_states, W_conf, b_conf, W_cls, b_cls, W_reg, b_reg):
    bsz, seq_len, hidden = hidden_states.shape
    x = hidden_states.reshape(bsz * seq_len, hidden)
    n = bsz * seq_len

    w = jnp.concatenate([W_conf, W_cls, W_reg], axis=1)
    b = jnp.concatenate([b_conf, b_cls, b_reg], axis=0).reshape(1, -1)
    c = w.shape[1]

    out = pl.pallas_call(
        _fused_heads_kernel,
        grid=(n // BLK,),
        in_specs=[
            pl.BlockSpec((BLK, hidden), lambda i: (i, 0)),
            pl.BlockSpec((hidden, c), lambda i: (0, 0)),
            pl.BlockSpec((1, c), lambda i: (0, 0)),
        ],
        out_specs=pl.BlockSpec((BLK, c), lambda i: (i, 0)),
        out_shape=jax.ShapeDtypeStruct((n, c), jnp.float32),
    )(x, w, b)

    conf = out[:, :NUM_ANCHORS].reshape(bsz, seq_len, NUM_ANCHORS)
    cls_ = out[:, NUM_ANCHORS:NUM_ANCHORS + NUM_ANCHORS * NUM_LABELS].reshape(
        bsz, seq_len, NUM_ANCHORS, NUM_LABELS
    )
    reg = out[:, NUM_ANCHORS + NUM_ANCHORS * NUM_LABELS:].reshape(
        bsz, seq_len, NUM_ANCHORS, 2
    )
    return (conf, cls_, reg)
